# baseline (speedup 1.0000x reference)
import jax
import jax.numpy as jnp
from jax import lax
from jax.experimental import pallas as pl
from jax.experimental.pallas import tpu as pltpu
from jax.experimental.pallas import tpu_sc as plsc

_B = 4
_HW = 256
_S = 16
_C = 133
_CP = 144
_TILE = 1024
_NC = 2
_NS = 16
_NW = _NC * _NS
_JOBS_PER_W = (_B * _S) // _NW


def _sample_constants():
    sel = jnp.full((_B, _HW), 1.0, jnp.float32) + 1e-11
    sel = sel / sel.sum(axis=1, keepdims=True)
    sel = jnp.log(sel)
    u = jax.random.uniform(jax.random.key(42), sel.shape, minval=1e-9, maxval=1.0)
    gumbel = -jnp.log(-jnp.log(u))
    _, mask_idx = jax.lax.top_k(sel + gumbel, _S)
    job_rows = (
        jnp.arange(_B, dtype=jnp.int32)[:, None] * _HW + mask_idx
    ).reshape(_NW, _JOBS_PER_W)
    tbl = jnp.concatenate(
        [job_rows, jnp.tile(job_rows[:, :1], (1, 16 - _JOBS_PER_W))], axis=1
    ).astype(jnp.int32)
    return mask_idx, tbl


def _sc_body(rows_hbm, tbl_hbm, out_hbm, idx_v, tile_v, hist_v, oh_v, sem):
    wid = lax.axis_index("c") * _NS + lax.axis_index("s")
    pltpu.sync_copy(tbl_hbm.at[wid], idx_v)
    idx_vec = idx_v[...]

    lane = lax.iota(jnp.int32, 16)
    ones = jnp.ones((16,), jnp.int32)
    lane_base = lane * _CP

    lane16 = lax.iota(jnp.int32, 16)

    copies = []
    col32 = []
    for k in range(_JOBS_PER_W):
        row = idx_vec[k]
        b = row >> 8
        t = row & 255
        col32.append(((t & 15) & 3) * 32)
        base = b * 2048 + (t >> 4) * 128 + ((t & 15) >> 2)
        idx_a = base + lane16 * 4
        copies.append(
            pltpu.async_copy(rows_hbm.at[idx_a], tile_v.at[k, pl.ds(0, 16)], sem)
        )
        copies.append(
            pltpu.async_copy(rows_hbm.at[idx_a + 64], tile_v.at[k, pl.ds(16, 16)], sem)
        )

    for k in range(_JOBS_PER_W):
        zeros16 = jnp.zeros((16,), jnp.int32)
        for i in range((16 * _CP) // 16):
            hist_v[pl.ds(i * 16, 16)] = zeros16

        copies[2 * k].wait()
        copies[2 * k + 1].wait()

        c32 = col32[k]
        for i in range(_TILE // 16):
            v = tile_v[k, i >> 1, pl.ds(c32 + (i & 1) * 16, 16)]
            plsc.addupdate_scatter(hist_v, [lane_base + v], ones)

        kmax = jnp.full((16,), -1, jnp.int32)
        for cb in range(_CP // 16):
            counts = hist_v[pl.ds(cb * 16, 16)]
            for l in range(1, 16):
                counts = counts + hist_v[pl.ds(l * _CP + cb * 16, 16)]
            key = counts * 2048 + (2047 - (lane + cb * 16))
            kmax = jnp.maximum(kmax, key)
        best = jnp.max(kmax)
        cls = 2047 - (best & 2047)

        for cb in range(_CP // 16):
            oh_v[pl.ds(cb * 16, 16)] = jnp.where(
                lane + cb * 16 == cls, 1.0, 0.0
            ).astype(jnp.float32)
        pltpu.sync_copy(oh_v, out_hbm.at[wid * _JOBS_PER_W + k])


@jax.jit
def _sc_call(rows2d, tbl):
    return pl.kernel(
        _sc_body,
        out_type=jax.ShapeDtypeStruct((_B * _S, _CP), jnp.float32),
        mesh=plsc.VectorSubcoreMesh(
            core_axis_name="c", subcore_axis_name="s",
            num_cores=_NC, num_subcores=_NS,
        ),
        scratch_types=[
            pltpu.VMEM((16,), jnp.int32),
            pltpu.VMEM((_JOBS_PER_W, 32, 128), jnp.int32),
            pltpu.VMEM((16 * _CP,), jnp.int32),
            pltpu.VMEM((_CP,), jnp.float32),
            pltpu.SemaphoreType.DMA,
        ],
        compiler_params=pltpu.CompilerParams(needs_layout_passes=False),
    )(rows2d, tbl)


def kernel(masks):
    mask_idx, tbl = _sample_constants()
    out = _sc_call(masks.reshape(_B * 512 * 4, 128), tbl)
    sampled = out[:, :_C].reshape(_B, _S, _C)
    return sampled, mask_idx


# speedup vs baseline: 1.1831x; 1.1831x over previous
import jax
import jax.numpy as jnp
from jax import lax
from jax.experimental import pallas as pl
from jax.experimental.pallas import tpu as pltpu
from jax.experimental.pallas import tpu_sc as plsc

_B = 4
_HW = 256
_S = 16
_C = 133
_CP = 144
_TILE = 1024
_NC = 2
_NS = 16
_NW = _NC * _NS
_JOBS_PER_W = (_B * _S) // _NW


def _sample_constants():
    sel = jnp.full((_B, _HW), 1.0, jnp.float32) + 1e-11
    sel = sel / sel.sum(axis=1, keepdims=True)
    sel = jnp.log(sel)
    u = jax.random.uniform(jax.random.key(42), sel.shape, minval=1e-9, maxval=1.0)
    gumbel = -jnp.log(-jnp.log(u))
    _, mask_idx = jax.lax.top_k(sel + gumbel, _S)
    job_rows = (
        jnp.arange(_B, dtype=jnp.int32)[:, None] * _HW + mask_idx
    ).reshape(_NW, _JOBS_PER_W)
    tbl = jnp.concatenate(
        [job_rows, jnp.tile(job_rows[:, :1], (1, 16 - _JOBS_PER_W))], axis=1
    ).astype(jnp.int32)
    return mask_idx, tbl


def _sc_body(rows_hbm, tbl_hbm, out_hbm, idx_v, tile_v, hist_v, oh_v, sem):
    wid = lax.axis_index("c") * _NS + lax.axis_index("s")
    pltpu.sync_copy(tbl_hbm.at[wid], idx_v)
    idx_vec = idx_v[...]

    lane = lax.iota(jnp.int32, 16)
    ones = jnp.ones((16,), jnp.int32)
    lane_base = lane * _CP

    copies = []
    col32 = []
    for k in range(_JOBS_PER_W):
        row = idx_vec[k]
        b = row >> 8
        t = row & 255
        col32.append((t & 3) * 32)
        copies.append(
            pltpu.async_copy(
                rows_hbm.at[
                    b, pl.ds((t >> 4) * 32, 32), pl.ds(((t & 15) >> 2) * 128, 128)
                ],
                tile_v.at[k],
                sem,
            )
        )

    for k in range(_JOBS_PER_W):
        zeros16 = jnp.zeros((16,), jnp.int32)
        for i in range((16 * _CP) // 16):
            hist_v[pl.ds(i * 16, 16)] = zeros16

        copies[k].wait()

        c32 = col32[k]
        for i in range(_TILE // 16):
            v = tile_v[k, i >> 1, pl.ds(c32 + (i & 1) * 16, 16)]
            plsc.addupdate_scatter(hist_v, [lane_base + v], ones)

        kmax = jnp.full((16,), -1, jnp.int32)
        for cb in range(_CP // 16):
            counts = hist_v[pl.ds(cb * 16, 16)]
            for l in range(1, 16):
                counts = counts + hist_v[pl.ds(l * _CP + cb * 16, 16)]
            key = counts * 2048 + (2047 - (lane + cb * 16))
            kmax = jnp.maximum(kmax, key)
        best = jnp.max(kmax)
        cls = 2047 - (best & 2047)

        for cb in range(_CP // 16):
            oh_v[pl.ds(cb * 16, 16)] = jnp.where(
                lane + cb * 16 == cls, 1.0, 0.0
            ).astype(jnp.float32)
        pltpu.sync_copy(oh_v, out_hbm.at[wid * _JOBS_PER_W + k])


@jax.jit
def _sc_call(rows2d, tbl):
    return pl.kernel(
        _sc_body,
        out_type=jax.ShapeDtypeStruct((_B * _S, _CP), jnp.float32),
        mesh=plsc.VectorSubcoreMesh(
            core_axis_name="c", subcore_axis_name="s",
            num_cores=_NC, num_subcores=_NS,
        ),
        scratch_types=[
            pltpu.VMEM((16,), jnp.int32),
            pltpu.VMEM((_JOBS_PER_W, 32, 128), jnp.int32),
            pltpu.VMEM((16 * _CP,), jnp.int32),
            pltpu.VMEM((_CP,), jnp.float32),
            pltpu.SemaphoreType.DMA,
        ],
        compiler_params=pltpu.CompilerParams(needs_layout_passes=False),
    )(rows2d, tbl)


def kernel(masks):
    mask_idx, tbl = _sample_constants()
    out = _sc_call(masks.reshape(_B, 512, 512), tbl)
    sampled = out[:, :_C].reshape(_B, _S, _C)
    return sampled, mask_idx
